# baseline (speedup 1.0000x reference)
import jax
import jax.numpy as jnp
from jax import lax
from jax.experimental import pallas as pl

NQ = 4
K = 1024
D = 256
BN = 1024


def _rvq_block(x_ref, cb_ref, q_ref, idx_ref, loss_ref):
    step = pl.program_id(0)

    @pl.when(step == 0)
    def _init():
        loss_ref[...] = jnp.zeros_like(loss_ref)

    r0 = x_ref[...]
    r = r0
    qsum = jnp.zeros((BN, D), jnp.float32)
    iota_k = lax.broadcasted_iota(jnp.int32, (BN, K), 1)
    idx_cols = lax.broadcasted_iota(jnp.int32, (BN, 8), 1)
    ones_row = jnp.ones((8, D), jnp.float32)
    idx_acc = jnp.zeros((BN, 8), jnp.int32)
    loss_rows = lax.broadcasted_iota(jnp.int32, (8, 128), 0)
    loss_cols = lax.broadcasted_iota(jnp.int32, (8, 128), 1)
    loss_contrib = jnp.zeros((8, 128), jnp.float32)

    for s in range(NQ):
        cb = cb_ref[s]
        rn = jnp.sum(r * r, axis=1, keepdims=True)
        cn = lax.dot_general(ones_row, cb * cb,
                             (((1,), (1,)), ((), ())),
                             preferred_element_type=jnp.float32,
                             precision=lax.Precision.HIGHEST)
        m2 = lax.dot_general(2.0 * r, cb, (((1,), (1,)), ((), ())),
                             preferred_element_type=jnp.float32)
        d2 = (rn - m2) + cn[0:1, :]
        dmin = jnp.min(d2, axis=1, keepdims=True)
        idx = jnp.min(jnp.where(d2 == dmin, iota_k, K), axis=1,
                      keepdims=True)
        onehot = (iota_k == idx).astype(jnp.float32)
        q = lax.dot_general(onehot, cb, (((1,), (0,)), ((), ())),
                            preferred_element_type=jnp.float32,
                            precision=lax.Precision.HIGHEST)
        r = r - q
        qsum = qsum + q
        loss_s = jnp.sum(r * r)
        loss_contrib = loss_contrib + loss_s * jnp.where(
            (loss_rows == 0) & (loss_cols == s), 1.0, 0.0)
        idx_acc = idx_acc + jnp.where(idx_cols == s, idx, 0)

    q_ref[...] = r0 + (qsum - r0)
    idx_ref[...] = idx_acc
    loss_ref[...] += loss_contrib


def kernel(x, codebooks):
    B, T, Dd = x.shape
    N = B * T
    flat = x.reshape(N, Dd)
    q_flat, idx_pack, loss_sums = pl.pallas_call(
        _rvq_block,
        grid=(N // BN,),
        in_specs=[
            pl.BlockSpec((BN, D), lambda i: (i, 0)),
            pl.BlockSpec((NQ, K, D), lambda i: (0, 0, 0)),
        ],
        out_specs=[
            pl.BlockSpec((BN, D), lambda i: (i, 0)),
            pl.BlockSpec((BN, 8), lambda i: (i, 0)),
            pl.BlockSpec((8, 128), lambda i: (0, 0)),
        ],
        out_shape=[
            jax.ShapeDtypeStruct((N, D), jnp.float32),
            jax.ShapeDtypeStruct((N, 8), jnp.int32),
            jax.ShapeDtypeStruct((8, 128), jnp.float32),
        ],
    )(flat, codebooks)
    quantized = q_flat.reshape(B, T, Dd)
    indices = idx_pack[:, :NQ].reshape(B, T, NQ)
    vq_loss = 1.25 * jnp.sum(loss_sums[0, :NQ]) / jnp.float32(N * Dd)
    losses = jnp.full((NQ,), vq_loss, dtype=jnp.float32)
    return quantized, indices, losses


# speedup vs baseline: 1.5494x; 1.5494x over previous
import jax
import jax.numpy as jnp
from jax import lax
from jax.experimental import pallas as pl

NQ = 4
K = 1024
D = 256
BN = 1024


def _rvq_block(x_ref, cb_ref, q_ref, idx_ref, loss_ref):
    step = pl.program_id(0)

    @pl.when(step == 0)
    def _init():
        loss_ref[...] = jnp.zeros_like(loss_ref)

    r0 = x_ref[...]
    r = r0
    qsum = jnp.zeros((BN, D), jnp.float32)
    iota_k = lax.broadcasted_iota(jnp.int32, (BN, K), 1)
    idx_cols = lax.broadcasted_iota(jnp.int32, (BN, 8), 1)
    ones_row = jnp.ones((8, D), jnp.float32)
    idx_acc = jnp.zeros((BN, 8), jnp.int32)
    loss_rows = lax.broadcasted_iota(jnp.int32, (8, 128), 0)
    loss_cols = lax.broadcasted_iota(jnp.int32, (8, 128), 1)
    loss_contrib = jnp.zeros((8, 128), jnp.float32)

    for s in range(NQ):
        cb = cb_ref[s]
        rn = jnp.sum(r * r, axis=1, keepdims=True)
        cn = lax.dot_general(ones_row, cb * cb,
                             (((1,), (1,)), ((), ())),
                             preferred_element_type=jnp.float32,
                             precision=lax.Precision.HIGHEST)
        m2 = lax.dot_general(2.0 * r, cb, (((1,), (1,)), ((), ())),
                             preferred_element_type=jnp.float32)
        d2 = (rn - m2) + cn[0:1, :]
        idx = jnp.argmin(d2, axis=1)[:, None]
        onehot = (iota_k == idx).astype(jnp.float32)
        cb_hi = cb.astype(jnp.bfloat16).astype(jnp.float32)
        rem = cb - cb_hi
        cb_mid = rem.astype(jnp.bfloat16).astype(jnp.float32)
        cb_lo = rem - cb_mid
        dn = (((1,), (0,)), ((), ()))
        q = ((lax.dot_general(onehot, cb_hi, dn,
                              preferred_element_type=jnp.float32)
              + lax.dot_general(onehot, cb_mid, dn,
                                preferred_element_type=jnp.float32))
             + lax.dot_general(onehot, cb_lo, dn,
                               preferred_element_type=jnp.float32))
        r = r - q
        qsum = qsum + q
        loss_s = jnp.sum(r * r)
        loss_contrib = loss_contrib + loss_s * jnp.where(
            (loss_rows == 0) & (loss_cols == s), 1.0, 0.0)
        idx_acc = idx_acc + jnp.where(idx_cols == s, idx, 0)

    q_ref[...] = r0 + (qsum - r0)
    idx_ref[...] = idx_acc
    loss_ref[...] += loss_contrib


def kernel(x, codebooks):
    B, T, Dd = x.shape
    N = B * T
    flat = x.reshape(N, Dd)
    q_flat, idx_pack, loss_sums = pl.pallas_call(
        _rvq_block,
        grid=(N // BN,),
        in_specs=[
            pl.BlockSpec((BN, D), lambda i: (i, 0)),
            pl.BlockSpec((NQ, K, D), lambda i: (0, 0, 0)),
        ],
        out_specs=[
            pl.BlockSpec((BN, D), lambda i: (i, 0)),
            pl.BlockSpec((BN, 8), lambda i: (i, 0)),
            pl.BlockSpec((8, 128), lambda i: (0, 0)),
        ],
        out_shape=[
            jax.ShapeDtypeStruct((N, D), jnp.float32),
            jax.ShapeDtypeStruct((N, 8), jnp.int32),
            jax.ShapeDtypeStruct((8, 128), jnp.float32),
        ],
    )(flat, codebooks)
    quantized = q_flat.reshape(B, T, Dd)
    indices = idx_pack[:, :NQ].reshape(B, T, NQ)
    vq_loss = 1.25 * jnp.sum(loss_sums[0, :NQ]) / jnp.float32(N * Dd)
    losses = jnp.full((NQ,), vq_loss, dtype=jnp.float32)
    return quantized, indices, losses


# speedup vs baseline: 1.5924x; 1.0277x over previous
import jax
import jax.numpy as jnp
from jax import lax
from jax.experimental import pallas as pl

NQ = 4
K = 1024
D = 256
BN = 1024


def _rvq_block(x_ref, cb_ref, q_ref, idx_ref, loss_ref):
    step = pl.program_id(0)

    @pl.when(step == 0)
    def _init():
        loss_ref[...] = jnp.zeros_like(loss_ref)

    r0 = x_ref[...]
    r = r0
    qsum = jnp.zeros((BN, D), jnp.float32)
    iota_k = lax.broadcasted_iota(jnp.int32, (BN, K), 1)
    idx_cols = lax.broadcasted_iota(jnp.int32, (BN, 8), 1)
    ones_row = jnp.ones((8, D), jnp.float32)
    idx_acc = jnp.zeros((BN, 8), jnp.int32)
    loss_rows = lax.broadcasted_iota(jnp.int32, (8, 128), 0)
    loss_cols = lax.broadcasted_iota(jnp.int32, (8, 128), 1)
    loss_contrib = jnp.zeros((8, 128), jnp.float32)

    for s in range(NQ):
        cb = cb_ref[s]
        rn = jnp.sum(r * r, axis=1, keepdims=True)
        cn = lax.dot_general(ones_row, cb * cb,
                             (((1,), (1,)), ((), ())),
                             preferred_element_type=jnp.float32,
                             precision=lax.Precision.HIGHEST)
        m2 = lax.dot_general(2.0 * r, cb, (((1,), (1,)), ((), ())),
                             preferred_element_type=jnp.float32)
        d2 = (rn - m2) + cn[0:1, :]
        dmin = jnp.min(d2, axis=1, keepdims=True)
        idx = jnp.min(jnp.where(d2 == dmin, iota_k, K), axis=1,
                      keepdims=True)
        onehot = (iota_k == idx).astype(jnp.float32)
        cb_hi = cb.astype(jnp.bfloat16).astype(jnp.float32)
        rem = cb - cb_hi
        cb_mid = rem.astype(jnp.bfloat16).astype(jnp.float32)
        cb_lo = rem - cb_mid
        dn = (((1,), (0,)), ((), ()))
        q = ((lax.dot_general(onehot, cb_hi, dn,
                              preferred_element_type=jnp.float32)
              + lax.dot_general(onehot, cb_mid, dn,
                                preferred_element_type=jnp.float32))
             + lax.dot_general(onehot, cb_lo, dn,
                               preferred_element_type=jnp.float32))
        r = r - q
        qsum = qsum + q
        loss_s = jnp.sum(r * r)
        loss_contrib = loss_contrib + loss_s * jnp.where(
            (loss_rows == 0) & (loss_cols == s), 1.0, 0.0)
        idx_acc = idx_acc + jnp.where(idx_cols == s, idx, 0)

    q_ref[...] = r0 + (qsum - r0)
    idx_ref[...] = idx_acc
    loss_ref[...] += loss_contrib


def kernel(x, codebooks):
    B, T, Dd = x.shape
    N = B * T
    flat = x.reshape(N, Dd)
    q_flat, idx_pack, loss_sums = pl.pallas_call(
        _rvq_block,
        grid=(N // BN,),
        in_specs=[
            pl.BlockSpec((BN, D), lambda i: (i, 0)),
            pl.BlockSpec((NQ, K, D), lambda i: (0, 0, 0)),
        ],
        out_specs=[
            pl.BlockSpec((BN, D), lambda i: (i, 0)),
            pl.BlockSpec((BN, 8), lambda i: (i, 0)),
            pl.BlockSpec((8, 128), lambda i: (0, 0)),
        ],
        out_shape=[
            jax.ShapeDtypeStruct((N, D), jnp.float32),
            jax.ShapeDtypeStruct((N, 8), jnp.int32),
            jax.ShapeDtypeStruct((8, 128), jnp.float32),
        ],
    )(flat, codebooks)
    quantized = q_flat.reshape(B, T, Dd)
    indices = idx_pack[:, :NQ].reshape(B, T, NQ)
    vq_loss = 1.25 * jnp.sum(loss_sums[0, :NQ]) / jnp.float32(N * Dd)
    losses = jnp.full((NQ,), vq_loss, dtype=jnp.float32)
    return quantized, indices, losses
